# baseline (speedup 1.0000x reference)
import jax
import jax.numpy as jnp
from jax import lax
from jax.experimental import pallas as pl
from jax.experimental.pallas import tpu as pltpu
from jax.experimental.pallas import tpu_sc as plsc

N = 10000
NPAD = 10240
E = 320000
EPAD = 327680
CHUNK = 128
NSUB = 16
BLK = 16
BN = 256
NB = NPAD // BN


def _mesh():
    return plsc.VectorSubcoreMesh(core_axis_name="c", subcore_axis_name="s")


def _fill_rows(ref, nrows, ncols, value):
    vals = jnp.full((16,), value, dtype=ref.dtype)

    def body(i, _):
        for j in range(ncols // 16):
            ref[i, pl.ds(j * 16, 16)] = vals
        return 0

    lax.fori_loop(0, nrows, body, 0)


def _zero_acc(stage_v, acc, row0, rows_per_tile):
    stage_rows = rows_per_tile // 16
    _fill_rows(stage_v, stage_rows, 128, 0.0)
    for k in range(16):
        pltpu.sync_copy(stage_v, acc.at[pl.ds(row0 + k * stage_rows, stage_rows)])


def _copy_out(stage_v, acc, out_hbm, row0, out_row0, rows_per_tile):
    stage_rows = rows_per_tile // 16

    def body(k, _):
        pltpu.sync_copy(acc.at[pl.ds(row0 + k * stage_rows, stage_rows)], stage_v)
        pltpu.sync_copy(stage_v, out_hbm.at[pl.ds(out_row0 + k * stage_rows,
                                                  stage_rows)])
        return 0

    lax.fori_loop(0, 16, body, 0)


def _hist_body(dst_hbm, out_hbm, dstb, ones_v, stage_v, acc, s0, s1):
    c = lax.axis_index("c")
    s = lax.axis_index("s")
    rows_per_tile = NPAD // NSUB
    row0 = s * rows_per_tile

    _zero_acc(stage_v, acc, row0, rows_per_tile)
    _fill_rows(ones_v, CHUNK, 128, 1.0)
    plsc.subcore_barrier()

    nblk = EPAD // 2 // NSUB // CHUNK // BLK
    tile = c * NSUB + s
    dummy = out_hbm.at[pl.ds(0, CHUNK)]

    def blk_body(blk, _):
        pltpu.sync_copy(dst_hbm.at[tile, pl.ds(blk * BLK, BLK)], dstb)
        pltpu.async_copy(ones_v, acc.at[dstb.at[0]], s0, add=True)
        pltpu.async_copy(ones_v, acc.at[dstb.at[1]], s1, add=True)

        def inner(m, _):
            pltpu.make_async_copy(dummy, ones_v, s0).wait()
            pltpu.async_copy(ones_v, acc.at[dstb.at[2 * m + 2]], s0, add=True)
            pltpu.make_async_copy(dummy, ones_v, s1).wait()
            pltpu.async_copy(ones_v, acc.at[dstb.at[2 * m + 3]], s1, add=True)
            return 0

        lax.fori_loop(0, BLK // 2 - 1, inner, 0)
        pltpu.make_async_copy(dummy, ones_v, s0).wait()
        pltpu.make_async_copy(dummy, ones_v, s1).wait()
        return 0

    lax.fori_loop(0, nblk, blk_body, 0)
    plsc.subcore_barrier()
    _copy_out(stage_v, acc, out_hbm, row0, c * NPAD + row0, rows_per_tile)


def _make_hist():
    return pl.kernel(
        _hist_body,
        out_type=jax.ShapeDtypeStruct((2 * NPAD, 128), jnp.float32),
        mesh=_mesh(),
        scratch_types=[
            pltpu.VMEM((BLK, CHUNK), jnp.int32),
            pltpu.VMEM((CHUNK, 128), jnp.float32),
            pltpu.VMEM((NPAD // NSUB // 16, 128), jnp.float32),
            pltpu.VMEM_SHARED((NPAD, 128), jnp.float32),
            pltpu.SemaphoreType.DMA,
            pltpu.SemaphoreType.DMA,
        ],
    )


def _agg_blocks(g_hbm, src_hbm, dst_hbm, tile, nblk, off,
                srcb, dstb, b0, b1, g0, g1, s0, s1, acc):
    dummy = g_hbm.at[pl.ds(0, CHUNK)]

    def blk_body(blk, _):
        pltpu.sync_copy(src_hbm.at[tile, pl.ds(blk * BLK, BLK)], srcb)
        pltpu.sync_copy(dst_hbm.at[tile, pl.ds(blk * BLK, BLK)], dstb)

        def ob(i, _):
            for j in range(CHUNK // 16):
                srcb[i, pl.ds(j * 16, 16)] = srcb[i, pl.ds(j * 16, 16)] + off
            return 0
        lax.fori_loop(0, BLK, ob, 0)

        pltpu.async_copy(g_hbm.at[srcb.at[0]], b0, g0)
        pltpu.async_copy(g_hbm.at[srcb.at[1]], b1, g1)

        def inner(m, _):
            pltpu.make_async_copy(dummy, b0, g0).wait()
            pltpu.async_copy(b0, acc.at[dstb.at[2 * m]], s0, add=True)

            @pl.when(2 * m + 2 < BLK)
            def _():
                pltpu.make_async_copy(dummy, b0, s0).wait()
                pltpu.async_copy(g_hbm.at[srcb.at[2 * m + 2]], b0, g0)

            pltpu.make_async_copy(dummy, b1, g1).wait()
            pltpu.async_copy(b1, acc.at[dstb.at[2 * m + 1]], s1, add=True)

            @pl.when(2 * m + 3 < BLK)
            def _():
                pltpu.make_async_copy(dummy, b1, s1).wait()
                pltpu.async_copy(g_hbm.at[srcb.at[2 * m + 3]], b1, g1)
            return 0

        lax.fori_loop(0, BLK // 2, inner, 0)
        pltpu.make_async_copy(dummy, b0, s0).wait()
        pltpu.make_async_copy(dummy, b1, s1).wait()
        return 0

    lax.fori_loop(0, nblk, blk_body, 0)


def _agg_split_body(g_hbm, src_hbm, dst_hbm, out_hbm,
                    srcb, dstb, b0, b1, stage_v, acc, g0, g1, s0, s1):
    c = lax.axis_index("c")
    s = lax.axis_index("s")
    rows_per_tile = NPAD // NSUB
    row0 = s * rows_per_tile

    _zero_acc(stage_v, acc, row0, rows_per_tile)
    plsc.subcore_barrier()

    nblk = EPAD // NSUB // CHUNK // BLK
    _agg_blocks(g_hbm, src_hbm, dst_hbm, s, nblk, c * NPAD,
                srcb, dstb, b0, b1, g0, g1, s0, s1, acc)
    plsc.subcore_barrier()
    _copy_out(stage_v, acc, out_hbm, row0, c * NPAD + row0, rows_per_tile)


def _agg_half_body(g_hbm, src_hbm, dst_hbm, out_hbm,
                   srcb, dstb, b0, b1, stage_v, acc, g0, g1, s0, s1):
    c = lax.axis_index("c")
    s = lax.axis_index("s")
    rows_per_tile = NPAD // NSUB
    row0 = s * rows_per_tile

    _zero_acc(stage_v, acc, row0, rows_per_tile)
    plsc.subcore_barrier()

    nblk = EPAD // 2 // NSUB // CHUNK // BLK
    zero_off = c * 0
    _agg_blocks(g_hbm, src_hbm, dst_hbm, c * NSUB + s, nblk, zero_off,
                srcb, dstb, b0, b1, g0, g1, s0, s1, acc)
    plsc.subcore_barrier()
    _copy_out(stage_v, acc, out_hbm, row0, c * NPAD + row0, rows_per_tile)


def _make_agg(body):
    return pl.kernel(
        body,
        out_type=jax.ShapeDtypeStruct((2 * NPAD, 128), jnp.float32),
        mesh=_mesh(),
        scratch_types=[
            pltpu.VMEM((BLK, CHUNK), jnp.int32),
            pltpu.VMEM((BLK, CHUNK), jnp.int32),
            pltpu.VMEM((CHUNK, 128), jnp.float32),
            pltpu.VMEM((CHUNK, 128), jnp.float32),
            pltpu.VMEM((NPAD // NSUB // 16, 128), jnp.float32),
            pltpu.VMEM_SHARED((NPAD, 128), jnp.float32),
            pltpu.SemaphoreType.DMA,
            pltpu.SemaphoreType.DMA,
            pltpu.SemaphoreType.DMA,
            pltpu.SemaphoreType.DMA,
        ],
    )


def _make_agg_split():
    return _make_agg(_agg_split_body)


def _make_agg64():
    return _make_agg(_agg_half_body)


def _tc_l1_body(x_ref, w_ref, ca_ref, cb_ref, g_ref, d_ref):
    d = lax.rsqrt(ca_ref[...] + cb_ref[...] + 1.0)
    h = jnp.dot(x_ref[...], w_ref[...], preferred_element_type=jnp.float32)
    g_ref[...] = h * d
    d_ref[...] = d


def _tc_mid_body(sa_ref, sb_ref, ga_ref, gb_ref, d_ref, b_ref, w_ref,
                 out_ref):
    d = d_ref[...]
    b = b_ref[...]
    xa = jnp.maximum(d * (sa_ref[...] + ga_ref[...]) + b[:, :128], 0.0)
    xb = jnp.maximum(d * (sb_ref[...] + gb_ref[...]) + b[:, 128:], 0.0)
    x = jnp.concatenate([xa, xb], axis=1)
    h = jnp.dot(x, w_ref[...], preferred_element_type=jnp.float32)
    out_ref[...] = h * d


def _tc_out_body(sa_ref, sb_ref, g_ref, d_ref, b_ref, out_ref):
    out_ref[...] = d_ref[...] * (sa_ref[...] + sb_ref[...] + g_ref[...]) + b_ref[...]


@jax.jit
def _run(x, src, dst, W1, b1, W2, b2, W3, b3):
    srcp = jnp.concatenate([src, jnp.zeros((EPAD - E,), jnp.int32)])
    dstp = jnp.concatenate([dst, jnp.full((EPAD - E,), N, jnp.int32)])
    nch_a = EPAD // NSUB // CHUNK
    nch_b = EPAD // 2 // NSUB // CHUNK
    srcp3a = srcp.reshape(NSUB, nch_a, CHUNK)
    dstp3a = dstp.reshape(NSUB, nch_a, CHUNK)
    srcp3b = srcp.reshape(2 * NSUB, nch_b, CHUNK)
    dstp3b = dstp.reshape(2 * NSUB, nch_b, CHUNK)
    xp = jnp.pad(x, ((0, NPAD - N), (0, 0)))
    W3p = jnp.pad(W3, ((0, 0), (0, 128 - 40)))
    b1r = b1.reshape(1, 256)
    b2r = b2.reshape(1, 256)
    b3r = jnp.pad(b3, (0, 128 - 40)).reshape(1, 128)

    cnt = _make_hist()(dstp3b)
    cA = cnt[:NPAD]
    cB = cnt[NPAD:]

    rowA = pl.BlockSpec((BN, 128), lambda j, i: (i, 0))
    rowB = pl.BlockSpec((BN, 128), lambda j, i: (NB + i, 0))
    out_split = pl.BlockSpec((BN, 128), lambda j, i: (j * NB + i, 0))

    g1, dinv = pl.pallas_call(
        _tc_l1_body,
        grid=(2, NB),
        in_specs=[
            pl.BlockSpec((BN, 128), lambda j, i: (i, 0)),
            pl.BlockSpec((128, 128), lambda j, i: (0, j)),
            rowA,
            rowA,
        ],
        out_specs=[out_split, rowA],
        out_shape=[jax.ShapeDtypeStruct((2 * NPAD, 128), jnp.float32),
                   jax.ShapeDtypeStruct((NPAD, 128), jnp.float32)],
    )(xp, W1, cA, cB)

    s1 = _make_agg_split()(g1, srcp3a, dstp3a)

    g2 = pl.pallas_call(
        _tc_mid_body,
        grid=(2, NB),
        in_specs=[
            rowA, rowB, rowA, rowB, rowA,
            pl.BlockSpec((1, 256), lambda j, i: (0, 0)),
            pl.BlockSpec((256, 128), lambda j, i: (0, j)),
        ],
        out_specs=out_split,
        out_shape=jax.ShapeDtypeStruct((2 * NPAD, 128), jnp.float32),
    )(s1, s1, g1, g1, dinv, b1r, W2)

    s2 = _make_agg_split()(g2, srcp3a, dstp3a)

    rowA1 = pl.BlockSpec((BN, 128), lambda i: (i, 0))
    rowB1 = pl.BlockSpec((BN, 128), lambda i: (NB + i, 0))

    g3 = pl.pallas_call(
        _tc_mid_body,
        grid=(NB,),
        in_specs=[
            rowA1, rowB1, rowA1, rowB1, rowA1,
            pl.BlockSpec((1, 256), lambda i: (0, 0)),
            pl.BlockSpec((256, 128), lambda i: (0, 0)),
        ],
        out_specs=pl.BlockSpec((BN, 128), lambda i: (i, 0)),
        out_shape=jax.ShapeDtypeStruct((NPAD, 128), jnp.float32),
    )(s2, s2, g2, g2, dinv, b2r, W3p)

    s3 = _make_agg64()(g3, srcp3b, dstp3b)

    out = pl.pallas_call(
        _tc_out_body,
        grid=(NB,),
        in_specs=[
            rowA1, rowB1, rowA1, rowA1,
            pl.BlockSpec((1, 128), lambda i: (0, 0)),
        ],
        out_specs=pl.BlockSpec((BN, 128), lambda i: (i, 0)),
        out_shape=jax.ShapeDtypeStruct((NPAD, 128), jnp.float32),
    )(s3, s3, g3, dinv, b3r)

    return out[:N, :40]


def kernel(x, edge_index, edge_weight, W1, b1, W2, b2, W3, b3):
    del edge_weight
    return _run(x, edge_index[0], edge_index[1], W1, b1, W2, b2, W3, b3)


# speedup vs baseline: 1.0766x; 1.0766x over previous
import jax
import jax.numpy as jnp
from jax import lax
from jax.experimental import pallas as pl
from jax.experimental.pallas import tpu as pltpu
from jax.experimental.pallas import tpu_sc as plsc

N = 10000
NPAD = 10240
E = 320000
EPAD = 327680
CHUNK = 128
NSUB = 16
BLK = 16
BN = 256
NB = NPAD // BN


def _mesh():
    return plsc.VectorSubcoreMesh(core_axis_name="c", subcore_axis_name="s")


def _fill_rows(ref, nrows, ncols, value):
    vals = jnp.full((16,), value, dtype=ref.dtype)

    def body(i, _):
        for j in range(ncols // 16):
            ref[i, pl.ds(j * 16, 16)] = vals
        return 0

    lax.fori_loop(0, nrows, body, 0)


def _zero_acc(stage_v, acc, row0, rows_per_tile):
    stage_rows = rows_per_tile // 16
    _fill_rows(stage_v, stage_rows, 128, 0.0)
    for k in range(16):
        pltpu.sync_copy(stage_v, acc.at[pl.ds(row0 + k * stage_rows, stage_rows)])


def _copy_out(stage_v, acc, out_hbm, row0, out_row0, rows_per_tile):
    stage_rows = rows_per_tile // 16

    def body(k, _):
        pltpu.sync_copy(acc.at[pl.ds(row0 + k * stage_rows, stage_rows)], stage_v)
        pltpu.sync_copy(stage_v, out_hbm.at[pl.ds(out_row0 + k * stage_rows,
                                                  stage_rows)])
        return 0

    lax.fori_loop(0, 16, body, 0)


def _hist_body(dst_hbm, out_hbm, dstb, ones_v, stage_v, acc, s0, s1):
    c = lax.axis_index("c")
    s = lax.axis_index("s")
    rows_per_tile = NPAD // NSUB
    row0 = s * rows_per_tile

    _zero_acc(stage_v, acc, row0, rows_per_tile)
    _fill_rows(ones_v, CHUNK, 128, 1.0)
    plsc.subcore_barrier()

    nblk = EPAD // 2 // NSUB // CHUNK // BLK
    tile = c * NSUB + s
    dummy = out_hbm.at[pl.ds(0, CHUNK)]

    def blk_body(blk, _):
        pltpu.sync_copy(dst_hbm.at[tile, pl.ds(blk * BLK, BLK)], dstb)
        pltpu.async_copy(ones_v, acc.at[dstb.at[0]], s0, add=True)
        pltpu.async_copy(ones_v, acc.at[dstb.at[1]], s1, add=True)

        def inner(m, _):
            pltpu.make_async_copy(dummy, ones_v, s0).wait()
            pltpu.async_copy(ones_v, acc.at[dstb.at[2 * m + 2]], s0, add=True)
            pltpu.make_async_copy(dummy, ones_v, s1).wait()
            pltpu.async_copy(ones_v, acc.at[dstb.at[2 * m + 3]], s1, add=True)
            return 0

        lax.fori_loop(0, BLK // 2 - 1, inner, 0)
        pltpu.make_async_copy(dummy, ones_v, s0).wait()
        pltpu.make_async_copy(dummy, ones_v, s1).wait()
        return 0

    lax.fori_loop(0, nblk, blk_body, 0)
    plsc.subcore_barrier()
    _copy_out(stage_v, acc, out_hbm, row0, c * NPAD + row0, rows_per_tile)


def _make_hist():
    return pl.kernel(
        _hist_body,
        out_type=jax.ShapeDtypeStruct((2 * NPAD, 128), jnp.float32),
        mesh=_mesh(),
        scratch_types=[
            pltpu.VMEM((BLK, CHUNK), jnp.int32),
            pltpu.VMEM((CHUNK, 128), jnp.float32),
            pltpu.VMEM((NPAD // NSUB // 16, 128), jnp.float32),
            pltpu.VMEM_SHARED((NPAD, 128), jnp.float32),
            pltpu.SemaphoreType.DMA,
            pltpu.SemaphoreType.DMA,
        ],
    )


def _agg_body(g_hbm, src_hbm, dst_hbm, out_hbm,
              srcb, dstb, b0, b1, stage_v, acc, g0, g1, s0, s1):
    c = lax.axis_index("c")
    s = lax.axis_index("s")
    rows_per_tile = NPAD // NSUB
    row0 = s * rows_per_tile
    tile = c * NSUB + s
    nblk = src_hbm.shape[1] // BLK

    _zero_acc(stage_v, acc, row0, rows_per_tile)
    plsc.subcore_barrier()

    dummy = g_hbm.at[pl.ds(0, CHUNK)]

    def blk_body(blk, _):
        pltpu.sync_copy(src_hbm.at[tile, pl.ds(blk * BLK, BLK)], srcb)
        pltpu.sync_copy(dst_hbm.at[tile, pl.ds(blk * BLK, BLK)], dstb)

        pltpu.async_copy(g_hbm.at[srcb.at[0]], b0, g0)
        pltpu.async_copy(g_hbm.at[srcb.at[1]], b1, g1)

        def inner(m, _):
            pltpu.make_async_copy(dummy, b0, g0).wait()
            pltpu.async_copy(b0, acc.at[dstb.at[2 * m]], s0, add=True)

            @pl.when(2 * m + 2 < BLK)
            def _():
                pltpu.make_async_copy(dummy, b0, s0).wait()
                pltpu.async_copy(g_hbm.at[srcb.at[2 * m + 2]], b0, g0)

            pltpu.make_async_copy(dummy, b1, g1).wait()
            pltpu.async_copy(b1, acc.at[dstb.at[2 * m + 1]], s1, add=True)

            @pl.when(2 * m + 3 < BLK)
            def _():
                pltpu.make_async_copy(dummy, b1, s1).wait()
                pltpu.async_copy(g_hbm.at[srcb.at[2 * m + 3]], b1, g1)
            return 0

        lax.fori_loop(0, BLK // 2, inner, 0)
        pltpu.make_async_copy(dummy, b0, s0).wait()
        pltpu.make_async_copy(dummy, b1, s1).wait()
        return 0

    lax.fori_loop(0, nblk, blk_body, 0)
    plsc.subcore_barrier()
    _copy_out(stage_v, acc, out_hbm, row0, c * NPAD + row0, rows_per_tile)


def _make_agg(nch):
    return pl.kernel(
        _agg_body,
        out_type=jax.ShapeDtypeStruct((2 * NPAD, 128), jnp.float32),
        mesh=_mesh(),
        scratch_types=[
            pltpu.VMEM((BLK, CHUNK), jnp.int32),
            pltpu.VMEM((BLK, CHUNK), jnp.int32),
            pltpu.VMEM((CHUNK, 128), jnp.float32),
            pltpu.VMEM((CHUNK, 128), jnp.float32),
            pltpu.VMEM((NPAD // NSUB // 16, 128), jnp.float32),
            pltpu.VMEM_SHARED((NPAD, 128), jnp.float32),
            pltpu.SemaphoreType.DMA,
            pltpu.SemaphoreType.DMA,
            pltpu.SemaphoreType.DMA,
            pltpu.SemaphoreType.DMA,
        ],
    )


def _tc_l1_body(x_ref, w_ref, ca_ref, cb_ref, g_ref, d_ref):
    d = lax.rsqrt(ca_ref[...] + cb_ref[...] + 1.0)
    h = jnp.dot(x_ref[...], w_ref[...], preferred_element_type=jnp.float32)
    g_ref[...] = h * d
    d_ref[...] = d


def _tc_mid_body(sa_ref, sb_ref, ga_ref, gb_ref, d_ref, b_ref, w_ref,
                 out_ref):
    d = d_ref[...]
    b = b_ref[...]
    xa = jnp.maximum(d * (sa_ref[...] + ga_ref[...]) + b[:, :128], 0.0)
    xb = jnp.maximum(d * (sb_ref[...] + gb_ref[...]) + b[:, 128:], 0.0)
    x = jnp.concatenate([xa, xb], axis=1)
    h = jnp.dot(x, w_ref[...], preferred_element_type=jnp.float32)
    out_ref[...] = h * d


def _tc_out_body(sa_ref, sb_ref, g_ref, d_ref, b_ref, out_ref):
    out_ref[...] = d_ref[...] * (sa_ref[...] + sb_ref[...] + g_ref[...]) + b_ref[...]


@jax.jit
def _run(x, src, dst, W1, b1, W2, b2, W3, b3):
    srcp = jnp.concatenate([src, jnp.zeros((EPAD - E,), jnp.int32)])
    dstp = jnp.concatenate([dst, jnp.full((EPAD - E,), N, jnp.int32)])
    nch_a = EPAD // NSUB // CHUNK
    nch_b = EPAD // 2 // NSUB // CHUNK
    srcp3a = srcp.reshape(NSUB, nch_a, CHUNK)
    dstp3a = dstp.reshape(NSUB, nch_a, CHUNK)
    srcp3b = srcp.reshape(2 * NSUB, nch_b, CHUNK)
    dstp3b = dstp.reshape(2 * NSUB, nch_b, CHUNK)
    srcA = jnp.concatenate([srcp3a, srcp3a + NPAD], axis=0)
    dstA = jnp.concatenate([dstp3a, dstp3a], axis=0)
    srcB = jnp.concatenate([srcp3b[:NSUB], srcp3b[NSUB:] + NPAD], axis=0)
    dstB = dstp3b
    xp = jnp.pad(x, ((0, NPAD - N), (0, 0)))
    W3p = jnp.pad(W3, ((0, 0), (0, 128 - 40)))
    b1r = b1.reshape(1, 256)
    b2r = b2.reshape(1, 256)
    b3r = jnp.pad(b3, (0, 128 - 40)).reshape(1, 128)

    cnt = _make_hist()(dstp3b)
    cA = cnt[:NPAD]
    cB = cnt[NPAD:]

    rowA = pl.BlockSpec((BN, 128), lambda j, i: (i, 0))
    rowB = pl.BlockSpec((BN, 128), lambda j, i: (NB + i, 0))
    out_split = pl.BlockSpec((BN, 128), lambda j, i: (j * NB + i, 0))

    g1, dinv = pl.pallas_call(
        _tc_l1_body,
        grid=(2, NB),
        in_specs=[
            pl.BlockSpec((BN, 128), lambda j, i: (i, 0)),
            pl.BlockSpec((128, 128), lambda j, i: (0, j)),
            rowA,
            rowA,
        ],
        out_specs=[out_split, rowA],
        out_shape=[jax.ShapeDtypeStruct((2 * NPAD, 128), jnp.float32),
                   jax.ShapeDtypeStruct((NPAD, 128), jnp.float32)],
    )(xp, W1, cA, cB)

    s1 = _make_agg(nch_a)(g1, srcA, dstA)

    g2 = pl.pallas_call(
        _tc_mid_body,
        grid=(2, NB),
        in_specs=[
            rowA, rowB, rowA, rowB, rowA,
            pl.BlockSpec((1, 256), lambda j, i: (0, 0)),
            pl.BlockSpec((256, 128), lambda j, i: (0, j)),
        ],
        out_specs=out_split,
        out_shape=jax.ShapeDtypeStruct((2 * NPAD, 128), jnp.float32),
    )(s1, s1, g1, g1, dinv, b1r, W2)

    s2 = _make_agg(nch_a)(g2, srcA, dstA)

    rowA1 = pl.BlockSpec((BN, 128), lambda i: (i, 0))
    rowB1 = pl.BlockSpec((BN, 128), lambda i: (NB + i, 0))

    g3 = pl.pallas_call(
        _tc_mid_body,
        grid=(2, NB),
        in_specs=[
            rowA, rowB, rowA, rowB, rowA,
            pl.BlockSpec((1, 256), lambda j, i: (0, 0)),
            pl.BlockSpec((256, 128), lambda j, i: (0, 0)),
        ],
        out_specs=out_split,
        out_shape=jax.ShapeDtypeStruct((2 * NPAD, 128), jnp.float32),
    )(s2, s2, g2, g2, dinv, b2r, W3p)

    s3 = _make_agg(nch_b)(g3, srcB, dstB)

    out = pl.pallas_call(
        _tc_out_body,
        grid=(NB,),
        in_specs=[
            rowA1, rowB1, rowA1, rowA1,
            pl.BlockSpec((1, 128), lambda i: (0, 0)),
        ],
        out_specs=pl.BlockSpec((BN, 128), lambda i: (i, 0)),
        out_shape=jax.ShapeDtypeStruct((NPAD, 128), jnp.float32),
    )(s3, s3, g3[:NPAD], dinv, b3r)

    return out[:N, :40]


def kernel(x, edge_index, edge_weight, W1, b1, W2, b2, W3, b3):
    del edge_weight
    return _run(x, edge_index[0], edge_index[1], W1, b1, W2, b2, W3, b3)


# speedup vs baseline: 1.1019x; 1.0235x over previous
import jax
import jax.numpy as jnp
from jax import lax
from jax.experimental import pallas as pl
from jax.experimental.pallas import tpu as pltpu
from jax.experimental.pallas import tpu_sc as plsc

N = 10000
NPAD = 10240
E = 320000
EPAD = 327680
CHUNK = 128
NSUB = 16
BLK = 40
BN = 256
NB = NPAD // BN


def _mesh():
    return plsc.VectorSubcoreMesh(core_axis_name="c", subcore_axis_name="s")


def _fill_rows(ref, nrows, ncols, value):
    vals = jnp.full((16,), value, dtype=ref.dtype)

    def body(i, _):
        for j in range(ncols // 16):
            ref[i, pl.ds(j * 16, 16)] = vals
        return 0

    lax.fori_loop(0, nrows, body, 0)


def _zero_acc(stage_v, acc, row0, rows_per_tile):
    stage_rows = rows_per_tile // 16
    _fill_rows(stage_v, stage_rows, 128, 0.0)
    for k in range(16):
        pltpu.sync_copy(stage_v, acc.at[pl.ds(row0 + k * stage_rows, stage_rows)])


def _copy_out(stage_v, acc, out_hbm, row0, out_row0, rows_per_tile):
    stage_rows = rows_per_tile // 16

    def body(k, _):
        pltpu.sync_copy(acc.at[pl.ds(row0 + k * stage_rows, stage_rows)], stage_v)
        pltpu.sync_copy(stage_v, out_hbm.at[pl.ds(out_row0 + k * stage_rows,
                                                  stage_rows)])
        return 0

    lax.fori_loop(0, 16, body, 0)


def _hist_body(dst_hbm, out_hbm, dstb, ones_v, stage_v, acc, s0, s1):
    c = lax.axis_index("c")
    s = lax.axis_index("s")
    rows_per_tile = NPAD // NSUB
    row0 = s * rows_per_tile

    _zero_acc(stage_v, acc, row0, rows_per_tile)
    _fill_rows(ones_v, CHUNK, 128, 1.0)
    plsc.subcore_barrier()

    nblk = EPAD // 2 // NSUB // CHUNK // BLK
    tile = c * NSUB + s
    dummy = out_hbm.at[pl.ds(0, CHUNK)]

    def blk_body(blk, _):
        pltpu.sync_copy(dst_hbm.at[tile, pl.ds(blk * BLK, BLK)], dstb)
        pltpu.async_copy(ones_v, acc.at[dstb.at[0]], s0, add=True)
        pltpu.async_copy(ones_v, acc.at[dstb.at[1]], s1, add=True)

        def inner(m, _):
            pltpu.make_async_copy(dummy, ones_v, s0).wait()
            pltpu.async_copy(ones_v, acc.at[dstb.at[2 * m + 2]], s0, add=True)
            pltpu.make_async_copy(dummy, ones_v, s1).wait()
            pltpu.async_copy(ones_v, acc.at[dstb.at[2 * m + 3]], s1, add=True)
            return 0

        lax.fori_loop(0, BLK // 2 - 1, inner, 0)
        pltpu.make_async_copy(dummy, ones_v, s0).wait()
        pltpu.make_async_copy(dummy, ones_v, s1).wait()
        return 0

    lax.fori_loop(0, nblk, blk_body, 0)
    plsc.subcore_barrier()
    _copy_out(stage_v, acc, out_hbm, row0, c * NPAD + row0, rows_per_tile)


def _make_hist():
    return pl.kernel(
        _hist_body,
        out_type=jax.ShapeDtypeStruct((2 * NPAD, 128), jnp.float32),
        mesh=_mesh(),
        scratch_types=[
            pltpu.VMEM((BLK, CHUNK), jnp.int32),
            pltpu.VMEM((CHUNK, 128), jnp.float32),
            pltpu.VMEM((NPAD // NSUB // 16, 128), jnp.float32),
            pltpu.VMEM_SHARED((NPAD, 128), jnp.float32),
            pltpu.SemaphoreType.DMA,
            pltpu.SemaphoreType.DMA,
        ],
    )


def _agg_body(g_hbm, src_hbm, dst_hbm, out_hbm,
              srcb, dstb, b0, b1, stage_v, acc, g0, g1, s0, s1):
    c = lax.axis_index("c")
    s = lax.axis_index("s")
    rows_per_tile = NPAD // NSUB
    row0 = s * rows_per_tile
    tile = c * NSUB + s
    nblk = src_hbm.shape[1] // BLK

    _zero_acc(stage_v, acc, row0, rows_per_tile)
    plsc.subcore_barrier()

    dummy = g_hbm.at[pl.ds(0, CHUNK)]

    def blk_body(blk, _):
        pltpu.sync_copy(src_hbm.at[tile, pl.ds(blk * BLK, BLK)], srcb)
        pltpu.sync_copy(dst_hbm.at[tile, pl.ds(blk * BLK, BLK)], dstb)

        pltpu.async_copy(g_hbm.at[srcb.at[0]], b0, g0)
        pltpu.async_copy(g_hbm.at[srcb.at[1]], b1, g1)

        def inner(m, _):
            pltpu.make_async_copy(dummy, b0, g0).wait()
            pltpu.async_copy(b0, acc.at[dstb.at[2 * m]], s0, add=True)

            @pl.when(2 * m + 2 < BLK)
            def _():
                pltpu.make_async_copy(dummy, b0, s0).wait()
                pltpu.async_copy(g_hbm.at[srcb.at[2 * m + 2]], b0, g0)

            pltpu.make_async_copy(dummy, b1, g1).wait()
            pltpu.async_copy(b1, acc.at[dstb.at[2 * m + 1]], s1, add=True)

            @pl.when(2 * m + 3 < BLK)
            def _():
                pltpu.make_async_copy(dummy, b1, s1).wait()
                pltpu.async_copy(g_hbm.at[srcb.at[2 * m + 3]], b1, g1)
            return 0

        lax.fori_loop(0, BLK // 2, inner, 0)
        pltpu.make_async_copy(dummy, b0, s0).wait()
        pltpu.make_async_copy(dummy, b1, s1).wait()
        return 0

    lax.fori_loop(0, nblk, blk_body, 0)
    plsc.subcore_barrier()
    _copy_out(stage_v, acc, out_hbm, row0, c * NPAD + row0, rows_per_tile)


def _make_agg(nch):
    return pl.kernel(
        _agg_body,
        out_type=jax.ShapeDtypeStruct((2 * NPAD, 128), jnp.float32),
        mesh=_mesh(),
        scratch_types=[
            pltpu.VMEM((BLK, CHUNK), jnp.int32),
            pltpu.VMEM((BLK, CHUNK), jnp.int32),
            pltpu.VMEM((CHUNK, 128), jnp.float32),
            pltpu.VMEM((CHUNK, 128), jnp.float32),
            pltpu.VMEM((NPAD // NSUB // 16, 128), jnp.float32),
            pltpu.VMEM_SHARED((NPAD, 128), jnp.float32),
            pltpu.SemaphoreType.DMA,
            pltpu.SemaphoreType.DMA,
            pltpu.SemaphoreType.DMA,
            pltpu.SemaphoreType.DMA,
        ],
    )


def _tc_l1_body(x_ref, w_ref, ca_ref, cb_ref, g_ref, d_ref):
    d = lax.rsqrt(ca_ref[...] + cb_ref[...] + 1.0)
    h = jnp.dot(x_ref[...], w_ref[...], preferred_element_type=jnp.float32)
    g_ref[...] = h * d
    d_ref[...] = d


def _tc_mid_body(sa_ref, sb_ref, ga_ref, gb_ref, d_ref, b_ref, w_ref,
                 out_ref):
    d = d_ref[...]
    b = b_ref[...]
    xa = jnp.maximum(d * (sa_ref[...] + ga_ref[...]) + b[:, :128], 0.0)
    xb = jnp.maximum(d * (sb_ref[...] + gb_ref[...]) + b[:, 128:], 0.0)
    x = jnp.concatenate([xa, xb], axis=1)
    h = jnp.dot(x, w_ref[...], preferred_element_type=jnp.float32)
    out_ref[...] = h * d


def _tc_out_body(sa_ref, sb_ref, g_ref, d_ref, b_ref, out_ref):
    out_ref[...] = d_ref[...] * (sa_ref[...] + sb_ref[...] + g_ref[...]) + b_ref[...]


@jax.jit
def _run(x, src, dst, W1, b1, W2, b2, W3, b3):
    srcp = jnp.concatenate([src, jnp.zeros((EPAD - E,), jnp.int32)])
    dstp = jnp.concatenate([dst, jnp.full((EPAD - E,), N, jnp.int32)])
    nch_a = EPAD // NSUB // CHUNK
    nch_b = EPAD // 2 // NSUB // CHUNK
    srcp3a = srcp.reshape(NSUB, nch_a, CHUNK)
    dstp3a = dstp.reshape(NSUB, nch_a, CHUNK)
    srcp3b = srcp.reshape(2 * NSUB, nch_b, CHUNK)
    dstp3b = dstp.reshape(2 * NSUB, nch_b, CHUNK)
    srcA = jnp.concatenate([srcp3a, srcp3a + NPAD], axis=0)
    dstA = jnp.concatenate([dstp3a, dstp3a], axis=0)
    srcB = jnp.concatenate([srcp3b[:NSUB], srcp3b[NSUB:] + NPAD], axis=0)
    dstB = dstp3b
    xp = jnp.pad(x, ((0, NPAD - N), (0, 0)))
    W3p = jnp.pad(W3, ((0, 0), (0, 128 - 40)))
    b1r = b1.reshape(1, 256)
    b2r = b2.reshape(1, 256)
    b3r = jnp.pad(b3, (0, 128 - 40)).reshape(1, 128)

    cnt = _make_hist()(dstp3b)
    cA = cnt[:NPAD]
    cB = cnt[NPAD:]

    rowA = pl.BlockSpec((BN, 128), lambda j, i: (i, 0))
    rowB = pl.BlockSpec((BN, 128), lambda j, i: (NB + i, 0))
    out_split = pl.BlockSpec((BN, 128), lambda j, i: (j * NB + i, 0))

    g1, dinv = pl.pallas_call(
        _tc_l1_body,
        grid=(2, NB),
        in_specs=[
            pl.BlockSpec((BN, 128), lambda j, i: (i, 0)),
            pl.BlockSpec((128, 128), lambda j, i: (0, j)),
            rowA,
            rowA,
        ],
        out_specs=[out_split, rowA],
        out_shape=[jax.ShapeDtypeStruct((2 * NPAD, 128), jnp.float32),
                   jax.ShapeDtypeStruct((NPAD, 128), jnp.float32)],
    )(xp, W1, cA, cB)

    s1 = _make_agg(nch_a)(g1, srcA, dstA)

    g2 = pl.pallas_call(
        _tc_mid_body,
        grid=(2, NB),
        in_specs=[
            rowA, rowB, rowA, rowB, rowA,
            pl.BlockSpec((1, 256), lambda j, i: (0, 0)),
            pl.BlockSpec((256, 128), lambda j, i: (0, j)),
        ],
        out_specs=out_split,
        out_shape=jax.ShapeDtypeStruct((2 * NPAD, 128), jnp.float32),
    )(s1, s1, g1, g1, dinv, b1r, W2)

    s2 = _make_agg(nch_a)(g2, srcA, dstA)

    rowA1 = pl.BlockSpec((BN, 128), lambda i: (i, 0))
    rowB1 = pl.BlockSpec((BN, 128), lambda i: (NB + i, 0))

    g3 = pl.pallas_call(
        _tc_mid_body,
        grid=(2, NB),
        in_specs=[
            rowA, rowB, rowA, rowB, rowA,
            pl.BlockSpec((1, 256), lambda j, i: (0, 0)),
            pl.BlockSpec((256, 128), lambda j, i: (0, 0)),
        ],
        out_specs=out_split,
        out_shape=jax.ShapeDtypeStruct((2 * NPAD, 128), jnp.float32),
    )(s2, s2, g2, g2, dinv, b2r, W3p)

    s3 = _make_agg(nch_b)(g3, srcB, dstB)

    out = pl.pallas_call(
        _tc_out_body,
        grid=(NB,),
        in_specs=[
            rowA1, rowB1, rowA1, rowA1,
            pl.BlockSpec((1, 128), lambda i: (0, 0)),
        ],
        out_specs=pl.BlockSpec((BN, 128), lambda i: (i, 0)),
        out_shape=jax.ShapeDtypeStruct((NPAD, 128), jnp.float32),
    )(s3, s3, g3[:NPAD], dinv, b3r)

    return out[:N, :40]


def kernel(x, edge_index, edge_weight, W1, b1, W2, b2, W3, b3):
    del edge_weight
    return _run(x, edge_index[0], edge_index[1], W1, b1, W2, b2, W3, b3)
